# baseline (speedup 1.0000x reference)
import functools

import jax
import jax.numpy as jnp
from jax import lax
from jax.experimental import pallas as pl
from jax.experimental.pallas import tpu as pltpu
from jax.experimental.pallas import tpu_sc as plsc

NC = 2
NS = 16
L = 16
K = 8


def _degree_body(np_, ept, ei_ref, hist_ref, idx_v, hist_v):
    c = lax.axis_index("c")
    s = lax.axis_index("s")
    row = 1 - c
    pltpu.sync_copy(ei_ref.at[row, pl.ds(s * ept, ept)], idx_v)
    zeros = jnp.zeros((L,), jnp.float32)
    ones = jnp.ones((L,), jnp.float32)

    def zero_body(i, _):
        hist_v[pl.ds(i * L, L)] = zeros
        return 0

    lax.fori_loop(0, np_ // L, zero_body, 0)

    def count_body(i, _):
        for u in range(4):
            iv = idx_v[pl.ds((i * 4 + u) * L, L)]
            plsc.addupdate_scatter(hist_v, [iv], ones)
        return 0

    lax.fori_loop(0, ept // (L * 4), count_body, 0)
    pltpu.sync_copy(hist_v, hist_ref.at[c, s])


def _gs_body(cpt, sg, rpt, ei2_ref, z_ref, x_ref, zrow_ref, ts_ref,
             idx_g, idx_s, *rest):
    bufs = rest[:K]
    acc = rest[K]
    gsem = rest[K + 1:2 * K + 1]
    ssem = rest[2 * K + 1:3 * K + 1]
    isem_g, isem_s = rest[3 * K + 1], rest[3 * K + 2]
    c = lax.axis_index("c")
    s = lax.axis_index("s")
    pltpu.sync_copy(zrow_ref.at[pl.ds(s * rpt, rpt)], acc.at[pl.ds(s * rpt, rpt)])
    plsc.subcore_barrier()

    nstages = cpt // sg

    def run(table_ref):
        def wait_g(b):
            for _ in range(2):
                pltpu.make_async_copy(table_ref.at[idx_g.at[0, 0]],
                                      bufs[b].at[0], gsem[b]).wait()

        def wait_s(b):
            for _ in range(2):
                pltpu.make_async_copy(bufs[b].at[0], acc.at[idx_s.at[0, 0]],
                                      ssem[b]).wait()

        def gather2(slot, b, q):
            pltpu.async_copy(table_ref.at[idx_g.at[slot, q]], bufs[b].at[0], gsem[b])
            pltpu.async_copy(table_ref.at[idx_g.at[slot, q + 1]], bufs[b].at[1], gsem[b])

        def scatter2(slot, b, q):
            pltpu.async_copy(bufs[b].at[0], acc.at[idx_s.at[slot, q]],
                             ssem[b], add=True)
            pltpu.async_copy(bufs[b].at[1], acc.at[idx_s.at[slot, q + 1]],
                             ssem[b], add=True)

        def load_idx(slot, st):
            base = s * cpt + st * sg
            pltpu.async_copy(ei2_ref.at[c, pl.ds(base, sg)], idx_g.at[slot], isem_g)
            pltpu.async_copy(ei2_ref.at[1 - c, pl.ds(base, sg)], idx_s.at[slot], isem_s)

        def wait_idx():
            pltpu.make_async_copy(ei2_ref.at[c, pl.ds(0, sg)], idx_g.at[0], isem_g).wait()
            pltpu.make_async_copy(ei2_ref.at[1 - c, pl.ds(0, sg)], idx_s.at[0], isem_s).wait()

        load_idx(0, 0)
        K2 = 2 * K

        def stage(st, _):
            slot = lax.rem(st, 2)
            wait_idx()

            @pl.when(st + 1 < nstages)
            def _():
                load_idx(1 - slot, st + 1)

            for b in range(K):
                gather2(slot, b, 2 * b)

            def group(i, _):
                jj = i * K2
                for b in range(K):
                    wait_g(b)
                    scatter2(slot, b, jj + 2 * b)
                for b in range(K):
                    wait_s(b)
                    gather2(slot, b, jj + K2 + 2 * b)
                return 0

            lax.fori_loop(0, sg // K2 - 1, group, 0)
            jj = sg - K2
            for b in range(K):
                wait_g(b)
                scatter2(slot, b, jj + 2 * b)
            for b in range(K):
                wait_s(b)
            return 0

        lax.fori_loop(0, nstages, stage, 0)

    @pl.when(c == 0)
    def _():
        run(z_ref)

    @pl.when(c == 1)
    def _():
        run(x_ref)

    plsc.subcore_barrier()
    pltpu.sync_copy(acc.at[pl.ds(s * rpt, rpt)], ts_ref.at[c, pl.ds(s * rpt, rpt)])


def _z_body(x_ref, w_ref, h0_ref, z_ref):
    indeg = jnp.sum(h0_ref[...], axis=0)
    dis = lax.rsqrt(indeg + 1.0)
    xw = jnp.dot(x_ref[...], w_ref[...], preferred_element_type=jnp.float32)
    z_ref[...] = xw * dis[:, None]


def _final_body(x_ref, s_ref, t_ref, z_ref, h0_ref, h1_ref, wd_ref,
                bg_ref, bd_ref, o_ref):
    indeg = jnp.sum(h0_ref[...], axis=0)
    cnt = jnp.sum(h1_ref[...], axis=0)
    dis = lax.rsqrt(indeg + 1.0)
    h = jnp.maximum(dis[:, None] * (t_ref[...] + z_ref[...]) + bg_ref[...], 0.0)
    proj = jnp.dot(x_ref[...] * s_ref[...], wd_ref[...],
                   preferred_element_type=jnp.float32)
    o_ref[...] = h + proj + cnt[:, None] * bd_ref[...]


def kernel(x, edge_index, W_gcn, b_gcn, W_diff, b_diff):
    N, D = x.shape
    E = edge_index.shape[1]

    chunk = 16
    sg = 32
    n_chunks = -(-E // (NS * sg * chunk)) * (NS * sg)
    EP = n_chunks * chunk
    cpt = n_chunks // NS
    ept = EP // NS
    NP = -(-(N + 1) // 128) * 128
    rpt = NP // NS

    ei = jnp.concatenate(
        [edge_index.astype(jnp.int32),
         jnp.full((2, EP - E), N, jnp.int32)], axis=1)
    ei2 = ei.reshape(2, n_chunks, chunk)
    x_p = jnp.concatenate([x, jnp.zeros((NP - N, D), x.dtype)], axis=0)
    zrow = jnp.zeros((NP, D), jnp.float32)

    mesh = plsc.VectorSubcoreMesh(core_axis_name="c", subcore_axis_name="s")
    sc_params = pltpu.CompilerParams(needs_layout_passes=False)

    hist = pl.kernel(
        functools.partial(_degree_body, NP, ept),
        out_type=jax.ShapeDtypeStruct((NC, NS, NP), jnp.float32),
        mesh=mesh,
        compiler_params=sc_params,
        scratch_types=[
            pltpu.VMEM((ept,), jnp.int32),
            pltpu.VMEM((NP,), jnp.float32),
        ],
    )(ei)
    h0 = hist[0]
    h1 = hist[1]

    z = pl.pallas_call(
        _z_body,
        out_shape=jax.ShapeDtypeStruct((NP, D), jnp.float32),
    )(x_p, W_gcn, h0)

    ts = pl.kernel(
        functools.partial(_gs_body, cpt, sg, rpt),
        out_type=jax.ShapeDtypeStruct((NC, NP, D), jnp.float32),
        mesh=mesh,
        compiler_params=sc_params,
        scratch_types=(
            [pltpu.VMEM((2, sg, chunk), jnp.int32)] * 2
            + [pltpu.VMEM((2, chunk, D), jnp.float32)] * K
            + [pltpu.VMEM_SHARED((NP, D), jnp.float32)]
            + [pltpu.SemaphoreType.DMA] * (2 * K + 2)
        ),
    )(ei2, z, x_p, zrow)

    out = pl.pallas_call(
        _final_body,
        out_shape=jax.ShapeDtypeStruct((NP, D), jnp.float32),
    )(x_p, ts[1], ts[0], z, h0, h1, W_diff,
      b_gcn.reshape(1, D), b_diff.reshape(1, D))

    return out[:N]


# speedup vs baseline: 1.3130x; 1.3130x over previous
import functools

import jax
import jax.numpy as jnp
from jax import lax
from jax.experimental import pallas as pl
from jax.experimental.pallas import tpu as pltpu
from jax.experimental.pallas import tpu_sc as plsc

NC = 2
NS = 16
L = 16
K = 8


def _degree_body(np_, ept, ei_ref, hist_ref, idx_v, hist_v):
    c = lax.axis_index("c")
    s = lax.axis_index("s")
    row = 1 - c
    pltpu.sync_copy(ei_ref.at[row, pl.ds(s * ept, ept)], idx_v)
    zeros = jnp.zeros((L,), jnp.float32)
    ones = jnp.ones((L,), jnp.float32)

    def zero_body(i, _):
        hist_v[pl.ds(i * L, L)] = zeros
        return 0

    lax.fori_loop(0, np_ // L, zero_body, 0)

    def count_body(i, _):
        for u in range(4):
            iv = idx_v[pl.ds((i * 4 + u) * L, L)]
            plsc.addupdate_scatter(hist_v, [iv], ones)
        return 0

    lax.fori_loop(0, ept // (L * 4), count_body, 0)
    pltpu.sync_copy(hist_v, hist_ref.at[c, s])


def _gs_body(cpt, sg, rpt, ei2_ref, z_ref, x_ref, zrow_ref, ts_ref,
             idx_g, idx_s, *rest):
    bufs = rest[:K]
    acc = rest[K]
    gsem = rest[K + 1:2 * K + 1]
    ssem = rest[2 * K + 1:3 * K + 1]
    isem_g, isem_s = rest[3 * K + 1], rest[3 * K + 2]
    c = lax.axis_index("c")
    s = lax.axis_index("s")
    pltpu.sync_copy(zrow_ref.at[pl.ds(s * rpt, rpt)], acc.at[pl.ds(s * rpt, rpt)])
    plsc.subcore_barrier()

    nstages = cpt // sg

    def run(table_ref):
        def wait_g(b):
            pltpu.make_async_copy(table_ref.at[idx_g.at[0, 0]], bufs[b], gsem[b]).wait()

        def wait_s(b):
            pltpu.make_async_copy(bufs[b], acc.at[idx_s.at[0, 0]], ssem[b]).wait()

        def load_idx(slot, st):
            base = s * cpt + st * sg
            pltpu.async_copy(ei2_ref.at[c, pl.ds(base, sg)], idx_g.at[slot], isem_g)
            pltpu.async_copy(ei2_ref.at[1 - c, pl.ds(base, sg)], idx_s.at[slot], isem_s)

        def wait_idx():
            pltpu.make_async_copy(ei2_ref.at[c, pl.ds(0, sg)], idx_g.at[0], isem_g).wait()
            pltpu.make_async_copy(ei2_ref.at[1 - c, pl.ds(0, sg)], idx_s.at[0], isem_s).wait()

        ngroups = cpt // K
        load_idx(0, 0)
        wait_idx()

        @pl.when(nstages > 1)
        def _():
            load_idx(1, 1)

        for b in range(K):
            pltpu.async_copy(table_ref.at[idx_g.at[0, b]], bufs[b], gsem[b])

        def group(g, _):
            st = (g * K) // sg
            slot = lax.rem(st, 2)
            q0 = lax.rem(g * K, sg)
            g1 = g + 1
            st1 = (g1 * K) // sg
            slot1 = lax.rem(st1, 2)
            q1 = lax.rem(g1 * K, sg)
            for b in range(K):
                wait_g(b)
                pltpu.async_copy(bufs[b], acc.at[idx_s.at[slot, q0 + b]],
                                 ssem[b], add=True)

            @pl.when(jnp.logical_and(g1 < ngroups, q1 == 0))
            def _():
                wait_idx()

                @pl.when(st1 + 1 < nstages)
                def _():
                    load_idx(1 - slot1, st1 + 1)

            @pl.when(g1 < ngroups)
            def _():
                for b in range(K):
                    wait_s(b)
                    pltpu.async_copy(table_ref.at[idx_g.at[slot1, q1 + b]],
                                     bufs[b], gsem[b])

            @pl.when(g1 == ngroups)
            def _():
                for b in range(K):
                    wait_s(b)

            return 0

        lax.fori_loop(0, ngroups, group, 0)

    @pl.when(c == 0)
    def _():
        run(z_ref)

    @pl.when(c == 1)
    def _():
        run(x_ref)

    plsc.subcore_barrier()
    pltpu.sync_copy(acc.at[pl.ds(s * rpt, rpt)], ts_ref.at[c, pl.ds(s * rpt, rpt)])


def _z_body(x_ref, w_ref, h0_ref, z_ref):
    indeg = jnp.sum(h0_ref[...], axis=0)
    dis = lax.rsqrt(indeg + 1.0)
    xw = jnp.dot(x_ref[...], w_ref[...], preferred_element_type=jnp.float32)
    z_ref[...] = xw * dis[:, None]


def _final_body(x_ref, s_ref, t_ref, z_ref, h0_ref, h1_ref, wd_ref,
                bg_ref, bd_ref, o_ref):
    indeg = jnp.sum(h0_ref[...], axis=0)
    cnt = jnp.sum(h1_ref[...], axis=0)
    dis = lax.rsqrt(indeg + 1.0)
    h = jnp.maximum(dis[:, None] * (t_ref[...] + z_ref[...]) + bg_ref[...], 0.0)
    proj = jnp.dot(x_ref[...] * s_ref[...], wd_ref[...],
                   preferred_element_type=jnp.float32)
    o_ref[...] = h + proj + cnt[:, None] * bd_ref[...]


def kernel(x, edge_index, W_gcn, b_gcn, W_diff, b_diff):
    N, D = x.shape
    E = edge_index.shape[1]

    chunk = 32
    sg = 32
    n_chunks = -(-E // (NS * sg * chunk)) * (NS * sg)
    EP = n_chunks * chunk
    cpt = n_chunks // NS
    ept = EP // NS
    NP = -(-(N + 1) // 128) * 128
    rpt = NP // NS

    ei = jnp.concatenate(
        [edge_index.astype(jnp.int32),
         jnp.full((2, EP - E), N, jnp.int32)], axis=1)
    ei2 = ei.reshape(2, n_chunks, chunk)
    x_p = jnp.concatenate([x, jnp.zeros((NP - N, D), x.dtype)], axis=0)
    zrow = jnp.zeros((NP, D), jnp.float32)

    mesh = plsc.VectorSubcoreMesh(core_axis_name="c", subcore_axis_name="s")
    sc_params = pltpu.CompilerParams(needs_layout_passes=False)

    hist = pl.kernel(
        functools.partial(_degree_body, NP, ept),
        out_type=jax.ShapeDtypeStruct((NC, NS, NP), jnp.float32),
        mesh=mesh,
        compiler_params=sc_params,
        scratch_types=[
            pltpu.VMEM((ept,), jnp.int32),
            pltpu.VMEM((NP,), jnp.float32),
        ],
    )(ei)
    h0 = hist[0]
    h1 = hist[1]

    z = pl.pallas_call(
        _z_body,
        out_shape=jax.ShapeDtypeStruct((NP, D), jnp.float32),
    )(x_p, W_gcn, h0)

    ts = pl.kernel(
        functools.partial(_gs_body, cpt, sg, rpt),
        out_type=jax.ShapeDtypeStruct((NC, NP, D), jnp.float32),
        mesh=mesh,
        compiler_params=sc_params,
        scratch_types=(
            [pltpu.VMEM((2, sg, chunk), jnp.int32)] * 2
            + [pltpu.VMEM((chunk, D), jnp.float32)] * K
            + [pltpu.VMEM_SHARED((NP, D), jnp.float32)]
            + [pltpu.SemaphoreType.DMA] * (2 * K + 2)
        ),
    )(ei2, z, x_p, zrow)

    out = pl.pallas_call(
        _final_body,
        out_shape=jax.ShapeDtypeStruct((NP, D), jnp.float32),
    )(x_p, ts[1], ts[0], z, h0, h1, W_diff,
      b_gcn.reshape(1, D), b_diff.reshape(1, D))

    return out[:N]
